# baseline (speedup 1.0000x reference)
import functools

import jax
import jax.numpy as jnp
from jax import lax
from jax.experimental import pallas as pl
from jax.experimental.pallas import tpu as pltpu
from jax.experimental.pallas import tpu_sc as plsc

_ROWS = 8192
_D = 1024
_NC = 2
_NS = 16
_NW = _NC * _NS
_LANES = 16

_SC_ROWS = 3072
_TC_ROWS = _ROWS - _SC_ROWS
_W_ROWS = _SC_ROWS // _NW
_CHUNK_ROWS = 16
_N_CHUNKS = _W_ROWS // _CHUNK_ROWS
_NBUF = 3


@functools.partial(
    pl.kernel,
    mesh=plsc.VectorSubcoreMesh(core_axis_name="c", subcore_axis_name="s"),
    out_type=jax.ShapeDtypeStruct((_SC_ROWS, _D), jnp.float32),
    scratch_types=(
        [pltpu.VMEM((_CHUNK_ROWS, _D), jnp.float32)] * _NBUF
        + [pltpu.VMEM((_CHUNK_ROWS, _D), jnp.float32)] * _NBUF
        + [pltpu.SemaphoreType.DMA] * (3 * _NBUF)
    ),
)
def _sc_add(x_hbm, pe_hbm, out_hbm, *scratch):
    xbufs = scratch[0:_NBUF]
    pebufs = scratch[_NBUF:2 * _NBUF]
    sx = scratch[2 * _NBUF:3 * _NBUF]
    sp = scratch[3 * _NBUF:4 * _NBUF]
    so = scratch[4 * _NBUF:5 * _NBUF]

    wid = lax.axis_index("s") * _NC + lax.axis_index("c")
    base = wid * _W_ROWS

    def fill(c):
        b = c % _NBUF
        off = base + c * _CHUNK_ROWS
        src = _TC_ROWS + off
        pltpu.async_copy(x_hbm.at[pl.ds(src, _CHUNK_ROWS)], xbufs[b], sx[b])
        pltpu.async_copy(pe_hbm.at[pl.ds(src, _CHUNK_ROWS)], pebufs[b], sp[b])

    def wait_fill(c):
        b = c % _NBUF
        off = base + c * _CHUNK_ROWS
        src = _TC_ROWS + off
        pltpu.make_async_copy(
            x_hbm.at[pl.ds(src, _CHUNK_ROWS)], xbufs[b], sx[b]).wait()
        pltpu.make_async_copy(
            pe_hbm.at[pl.ds(src, _CHUNK_ROWS)], pebufs[b], sp[b]).wait()

    def drain(c):
        b = c % _NBUF
        off = base + c * _CHUNK_ROWS
        pltpu.async_copy(xbufs[b], out_hbm.at[pl.ds(off, _CHUNK_ROWS)], so[b])

    def wait_drain(c):
        b = c % _NBUF
        off = base + c * _CHUNK_ROWS
        pltpu.make_async_copy(
            xbufs[b], out_hbm.at[pl.ds(off, _CHUNK_ROWS)], so[b]).wait()

    fill(0)
    fill(1)
    for c in range(_N_CHUNKS):
        if c + 2 < _N_CHUNKS:
            if c >= 1:
                wait_drain(c - 1)
            fill(c + 2)
        b = c % _NBUF
        wait_fill(c)

        def add_body(j, carry, b=b):
            s = pl.ds(j * _LANES, _LANES)
            for r in range(_CHUNK_ROWS):
                plsc.addupdate(xbufs[b].at[r, s], pebufs[b][r, s])
            return carry

        lax.fori_loop(0, _D // _LANES, add_body, 0)
        drain(c)
    for c in range(max(0, _N_CHUNKS - 3), _N_CHUNKS):
        wait_drain(c)


def _tc_add(x_ref, pe_ref, o_ref):
    o_ref[...] = x_ref[...] + pe_ref[...]


def kernel(x, pos_emb):
    seq_len, d = x.shape
    pe = pos_emb[:seq_len]
    sc_out = _sc_add(x, pe)
    blk = 1024
    tc_out = pl.pallas_call(
        _tc_add,
        grid=(_TC_ROWS // blk,),
        in_specs=[
            pl.BlockSpec((blk, d), lambda i: (i, 0)),
            pl.BlockSpec((blk, d), lambda i: (i, 0)),
        ],
        out_specs=pl.BlockSpec((blk, d), lambda i: (i, 0)),
        out_shape=jax.ShapeDtypeStruct((_TC_ROWS, d), x.dtype),
    )(x, pe)
    return jnp.concatenate([tc_out, sc_out], axis=0)


# speedup vs baseline: 2.5344x; 2.5344x over previous
import jax
import jax.numpy as jnp
from jax.experimental import pallas as pl


def _add_kernel(x_ref, pe_ref, o_ref):
    o_ref[...] = x_ref[...] + pe_ref[...]


def kernel(x, pos_emb):
    seq_len, d = x.shape
    blk = 1024
    grid = (seq_len // blk,)
    return pl.pallas_call(
        _add_kernel,
        grid=grid,
        in_specs=[
            pl.BlockSpec((blk, d), lambda i: (i, 0)),
            pl.BlockSpec((blk, d), lambda i: (i, 0)),
        ],
        out_specs=pl.BlockSpec((blk, d), lambda i: (i, 0)),
        out_shape=jax.ShapeDtypeStruct((seq_len, d), x.dtype),
    )(x, pos_emb[:seq_len])
